# baseline (speedup 1.0000x reference)
import functools

import jax
import jax.numpy as jnp
from jax.experimental import pallas as pl
from jax.experimental.pallas import tpu as pltpu

B, SEQ, D = 32, 196, 768
POOL, LENGTH, TOPK = 100, 10, 5


def _fused_kernel(x_embed_ref, x_key_ref, prompt_ref, prompt_key_ref,
                  out_ref, rs_ref):
    b = pl.program_id(0)

    xk = x_key_ref[pl.ds(b, 1), :]
    xk = xk / jnp.maximum(jnp.sqrt(jnp.sum(xk * xk)), 1e-12)

    pk = prompt_key_ref[...]
    pk = pk / jnp.maximum(
        jnp.sqrt(jnp.sum(pk * pk, axis=1, keepdims=True)), 1e-12)

    sim = jnp.dot(xk, pk.T, preferred_element_type=jnp.float32)

    m = jnp.max(sim)
    e = jnp.exp(sim - m)
    p = e / jnp.sum(e)

    mean_flat = jnp.dot(p, prompt_ref[...],
                        preferred_element_type=jnp.float32) * (1.0 / POOL)
    out_ref[0, :LENGTH, :] = mean_flat.reshape(LENGTH, D)
    out_ref[0, LENGTH:, :] = x_embed_ref[0]

    iota = jax.lax.broadcasted_iota(jnp.int32, (1, POOL), 1)
    v = sim
    acc = jnp.float32(0.0)
    for _ in range(TOPK):
        mx = jnp.max(v)
        idx = jnp.min(jnp.where(v >= mx, iota, jnp.int32(POOL)))
        acc = acc + mx
        v = jnp.where(iota == idx, -jnp.inf, v)

    prev = rs_ref[...]
    prev = jnp.where(b == 0, jnp.zeros_like(prev), prev)
    rs_ref[...] = prev + acc * (1.0 / B)


@jax.jit
def kernel(x_embed, x_key, prompt, prompt_key):
    prompt_flat = prompt.reshape(POOL, LENGTH * D)
    out, rs = pl.pallas_call(
        _fused_kernel,
        grid=(B,),
        in_specs=[
            pl.BlockSpec((1, SEQ, D), lambda b: (b, 0, 0)),
            pl.BlockSpec((B, 2 * D), lambda b: (0, 0)),
            pl.BlockSpec((POOL, LENGTH * D), lambda b: (0, 0)),
            pl.BlockSpec((POOL, 2 * D), lambda b: (0, 0)),
        ],
        out_specs=[
            pl.BlockSpec((1, LENGTH + SEQ, D), lambda b: (b, 0, 0)),
            pl.BlockSpec((1, 1), lambda b: (0, 0)),
        ],
        out_shape=[
            jax.ShapeDtypeStruct((B, LENGTH + SEQ, D), jnp.float32),
            jax.ShapeDtypeStruct((1, 1), jnp.float32),
        ],
    )(x_embed, x_key, prompt_flat, prompt_key)
    return out, rs[0, 0]
